# baseline (speedup 1.0000x reference)
import functools

import jax
import jax.numpy as jnp
from jax import lax
from jax.experimental import pallas as pl
from jax.experimental.pallas import tpu as pltpu
from jax.experimental.pallas import tpu_sc as plsc

N = 10000
E = 320000
D = 128
NCLS = 256
NC = 2
NS = 16
NW = NC * NS
CH = 128
NCHUNK = 81
EPW = NCHUNK * CH
PAD = EPW - E // NW
ACC_ROWS = 10112
TRASH = N
RPT = ACC_ROWS // NS

_mesh = plsc.VectorSubcoreMesh(core_axis_name="c", subcore_axis_name="s")


def _zero_vmem_rows(buf, nrows, ncols):
    z = jnp.zeros((16,), jnp.float32)

    def body(k, _):
        i = k // (ncols // 16)
        j = k % (ncols // 16)
        buf[i, pl.ds(j * 16, 16)] = z
        return 0

    lax.fori_loop(0, nrows * (ncols // 16), body, 0)


@functools.partial(
    pl.kernel,
    out_type=jax.ShapeDtypeStruct((NC, ACC_ROWS, 16), jnp.float32),
    mesh=_mesh,
    scratch_types=[
        pltpu.VMEM((CH,), jnp.int32),
        pltpu.VMEM((CH, 16), jnp.float32),
        pltpu.VMEM_SHARED((ACC_ROWS, 16), jnp.float32),
    ],
)
def _sc_degree(dst_hbm, out_hbm, didx, obuf, hist):
    c = lax.axis_index("c")
    s = lax.axis_index("s")
    w = s * NC + c

    _zero_vmem_rows(obuf, CH, 16)
    r0 = s * RPT
    pltpu.sync_copy(obuf.at[pl.ds(0, CH)], hist.at[pl.ds(r0, CH)])
    pltpu.sync_copy(obuf.at[pl.ds(0, CH)], hist.at[pl.ds(r0 + CH, CH)])
    pltpu.sync_copy(obuf.at[pl.ds(0, CH)], hist.at[pl.ds(r0 + 2 * CH, CH)])
    pltpu.sync_copy(obuf.at[pl.ds(0, CH)], hist.at[pl.ds(r0 + 3 * CH, CH)])
    pltpu.sync_copy(obuf.at[pl.ds(0, RPT - 4 * CH)],
                    hist.at[pl.ds(r0 + 4 * CH, RPT - 4 * CH)])

    one = jnp.ones((16,), jnp.float32)

    def fill(i, _):
        obuf[i, pl.ds(0, 16)] = one
        return 0

    lax.fori_loop(0, CH, fill, 0)
    plsc.subcore_barrier()

    def chunk(j, _):
        base = w * EPW + j * CH
        pltpu.sync_copy(dst_hbm.at[pl.ds(base, CH)], didx)
        pltpu.sync_copy(obuf, hist.at[didx], add=True)
        return 0

    lax.fori_loop(0, NCHUNK, chunk, 0)
    plsc.subcore_barrier()

    pltpu.sync_copy(hist.at[pl.ds(r0, RPT)], out_hbm.at[c, pl.ds(r0, RPT)])


@functools.partial(
    pl.kernel,
    out_type=jax.ShapeDtypeStruct((NC, ACC_ROWS, D), jnp.float32),
    mesh=_mesh,
    scratch_types=[
        pltpu.VMEM((2, CH), jnp.int32),
        pltpu.VMEM((2, CH), jnp.int32),
        pltpu.VMEM((2, CH), jnp.int32),
        pltpu.VMEM((CH, D), jnp.float32),
        pltpu.VMEM((CH, D), jnp.float32),
        pltpu.VMEM((CH, D), jnp.float32),
        pltpu.VMEM_SHARED((ACC_ROWS, D), jnp.float32),
        pltpu.SemaphoreType.DMA,
        pltpu.SemaphoreType.DMA,
        pltpu.SemaphoreType.DMA,
    ],
)
def _sc_scatter(g_hbm, idx_hbm, out_hbm, idx0, idx1, idx2,
                rows0, rows1, rows2, acc, gsem0, gsem1, gsem2):
    c = lax.axis_index("c")
    s = lax.axis_index("s")
    w = s * NC + c

    _zero_vmem_rows(rows0, CH, D)
    r0 = s * RPT
    pltpu.sync_copy(rows0.at[pl.ds(0, CH)], acc.at[pl.ds(r0, CH)])
    pltpu.sync_copy(rows0.at[pl.ds(0, CH)], acc.at[pl.ds(r0 + CH, CH)])
    pltpu.sync_copy(rows0.at[pl.ds(0, CH)], acc.at[pl.ds(r0 + 2 * CH, CH)])
    pltpu.sync_copy(rows0.at[pl.ds(0, CH)], acc.at[pl.ds(r0 + 3 * CH, CH)])
    pltpu.sync_copy(rows0.at[pl.ds(0, RPT - 4 * CH)],
                    acc.at[pl.ds(r0 + 4 * CH, RPT - 4 * CH)])
    plsc.subcore_barrier()

    cbase = w * NCHUNK

    pltpu.sync_copy(idx_hbm.at[cbase], idx0)
    pltpu.sync_copy(idx_hbm.at[cbase + 1], idx1)
    pltpu.sync_copy(idx_hbm.at[cbase + 2], idx2)
    pltpu.async_copy(g_hbm.at[idx0.at[0]], rows0, gsem0)
    pltpu.async_copy(g_hbm.at[idx1.at[0]], rows1, gsem1)
    pltpu.async_copy(g_hbm.at[idx2.at[0]], rows2, gsem2)
    ntri = NCHUNK // 3

    def body(i, _):
        for t, (idx, rows, gsem) in enumerate(
                ((idx0, rows0, gsem0),
                 (idx1, rows1, gsem1),
                 (idx2, rows2, gsem2))):
            j = 3 * i + t
            pltpu.make_async_copy(g_hbm.at[idx.at[0]], rows, gsem).wait()

            pltpu.sync_copy(rows, acc.at[idx.at[1]], add=True)

            @pl.when(i < ntri - 1)
            def _():
                pltpu.sync_copy(idx_hbm.at[cbase + j + 3], idx)
                pltpu.async_copy(g_hbm.at[idx.at[0]], rows, gsem)

        return 0

    lax.fori_loop(0, ntri, body, 0)
    plsc.subcore_barrier()

    pltpu.sync_copy(acc.at[pl.ds(r0, RPT)], out_hbm.at[c, pl.ds(r0, RPT)])


_RB = 1000
_GRID = N // _RB


def _tc_first_body(x_ref, w_ref, d0_ref, d1_ref, g_ref, dinv_ref):
    deg = 1.0 + d0_ref[:, 0] + d1_ref[:, 0]
    dinv = lax.rsqrt(deg)[:, None]
    g_ref[...] = jnp.dot(x_ref[...], w_ref[...],
                         preferred_element_type=jnp.float32) * dinv
    dinv_ref[...] = jnp.broadcast_to(dinv, dinv_ref.shape)


def _tc_first(x, w0, d0, d1):
    return pl.pallas_call(
        _tc_first_body,
        grid=(_GRID,),
        in_specs=[
            pl.BlockSpec((_RB, D), lambda i: (i, 0)),
            pl.BlockSpec((D, D), lambda i: (0, 0)),
            pl.BlockSpec((_RB, 16), lambda i: (i, 0)),
            pl.BlockSpec((_RB, 16), lambda i: (i, 0)),
        ],
        out_specs=[
            pl.BlockSpec((_RB, D), lambda i: (i, 0)),
            pl.BlockSpec((_RB, D), lambda i: (i, 0)),
        ],
        out_shape=[
            jax.ShapeDtypeStruct((N, D), jnp.float32),
            jax.ShapeDtypeStruct((N, D), jnp.float32),
        ],
    )(x, w0, d0, d1)


def _tc_mid_body(acc_ref, g_ref, dinv_ref, b_ref, w_ref, out_ref):
    a = acc_ref[0] + acc_ref[1] + g_ref[...]
    h = jnp.maximum(a * dinv_ref[...] + b_ref[...], 0.0)
    out_ref[...] = jnp.dot(h, w_ref[...],
                           preferred_element_type=jnp.float32) * dinv_ref[...]


def _tc_mid(accp, g, dinvb, b, w):
    return pl.pallas_call(
        _tc_mid_body,
        grid=(_GRID,),
        in_specs=[
            pl.BlockSpec((NC, _RB, D), lambda i: (0, i, 0)),
            pl.BlockSpec((_RB, D), lambda i: (i, 0)),
            pl.BlockSpec((_RB, D), lambda i: (i, 0)),
            pl.BlockSpec((1, D), lambda i: (0, 0)),
            pl.BlockSpec((D, D), lambda i: (0, 0)),
        ],
        out_specs=pl.BlockSpec((_RB, D), lambda i: (i, 0)),
        out_shape=jax.ShapeDtypeStruct((N, D), jnp.float32),
    )(accp, g, dinvb, b, w)


def _tc_final_body(acc_ref, g_ref, dinv_ref, b_ref, wfc_ref, bfc_ref, out_ref):
    a = acc_ref[0] + acc_ref[1] + g_ref[...]
    h = jnp.maximum(a * dinv_ref[...] + b_ref[...], 0.0)
    out_ref[...] = jnp.dot(h, wfc_ref[...],
                           preferred_element_type=jnp.float32) + bfc_ref[...]


def _tc_final(accp, g, dinvb, b, wfc, bfc):
    return pl.pallas_call(
        _tc_final_body,
        grid=(_GRID,),
        in_specs=[
            pl.BlockSpec((NC, _RB, D), lambda i: (0, i, 0)),
            pl.BlockSpec((_RB, D), lambda i: (i, 0)),
            pl.BlockSpec((_RB, D), lambda i: (i, 0)),
            pl.BlockSpec((1, D), lambda i: (0, 0)),
            pl.BlockSpec((D, NCLS), lambda i: (0, 0)),
            pl.BlockSpec((1, NCLS), lambda i: (0, 0)),
        ],
        out_specs=pl.BlockSpec((_RB, NCLS), lambda i: (i, 0)),
        out_shape=jax.ShapeDtypeStruct((N, NCLS), jnp.float32),
    )(accp, g, dinvb, b, wfc, bfc)


def kernel(x, edge_index, W0, b0, W1, b1, W2, b2, Wfc, bfc):
    src = edge_index[0].astype(jnp.int32)
    dst = edge_index[1].astype(jnp.int32)

    src_w = src.reshape(NW, E // NW)
    dst_w = dst.reshape(NW, E // NW)
    pad_s = jnp.zeros((NW, PAD), jnp.int32)
    pad_d = jnp.full((NW, PAD), TRASH, jnp.int32)
    src_w = jnp.concatenate([src_w, pad_s], axis=1)
    dst_w = jnp.concatenate([dst_w, pad_d], axis=1)
    dst_p = dst_w.reshape(-1)
    idx_p = jnp.stack([src_w.reshape(NW, NCHUNK, CH),
                       dst_w.reshape(NW, NCHUNK, CH)],
                      axis=2).reshape(NW * NCHUNK, 2, CH)

    degp = _sc_degree(dst_p)
    d0 = degp[0, :N, :]
    d1 = degp[1, :N, :]

    g0, dinvb = _tc_first(x, W0, d0, d1)

    acc1 = _sc_scatter(g0, idx_p)
    g1 = _tc_mid(acc1[:, :N, :], g0, dinvb, b0.reshape(1, D), W1)

    acc2 = _sc_scatter(g1, idx_p)
    g2 = _tc_mid(acc2[:, :N, :], g1, dinvb, b1.reshape(1, D), W2)

    acc3 = _sc_scatter(g2, idx_p)
    out = _tc_final(acc3[:, :N, :], g2, dinvb, b2.reshape(1, D), Wfc,
                    bfc.reshape(1, NCLS))
    return out


# speedup vs baseline: 1.2539x; 1.2539x over previous
import functools

import jax
import jax.numpy as jnp
from jax import lax
from jax.experimental import pallas as pl
from jax.experimental.pallas import tpu as pltpu
from jax.experimental.pallas import tpu_sc as plsc

N = 10000
E = 320000
D = 128
NCLS = 256
NC = 2
NS = 16
NW = NC * NS
CH = 128
EPW = 10240
NCHUNK = EPW // CH
PAD = EPW - E // NW
ACC_ROWS = 10112
TRASH = N
RPT = ACC_ROWS // NS

_mesh = plsc.VectorSubcoreMesh(core_axis_name="c", subcore_axis_name="s")


def _zero_vmem_rows(buf, nrows, ncols):
    z = jnp.zeros((16,), jnp.float32)

    def body(k, _):
        i = k // (ncols // 16)
        j = k % (ncols // 16)
        buf[i, pl.ds(j * 16, 16)] = z
        return 0

    lax.fori_loop(0, nrows * (ncols // 16), body, 0)


@functools.partial(
    pl.kernel,
    out_type=jax.ShapeDtypeStruct((NC, ACC_ROWS, 16), jnp.float32),
    mesh=_mesh,
    scratch_types=[
        pltpu.VMEM((CH,), jnp.int32),
        pltpu.VMEM((CH, 16), jnp.float32),
        pltpu.VMEM_SHARED((ACC_ROWS, 16), jnp.float32),
    ],
)
def _sc_degree(dst_hbm, out_hbm, didx, obuf, hist):
    c = lax.axis_index("c")
    s = lax.axis_index("s")
    w = s * NC + c

    _zero_vmem_rows(obuf, CH, 16)
    r0 = s * RPT
    pltpu.sync_copy(obuf.at[pl.ds(0, CH)], hist.at[pl.ds(r0, CH)])
    pltpu.sync_copy(obuf.at[pl.ds(0, CH)], hist.at[pl.ds(r0 + CH, CH)])
    pltpu.sync_copy(obuf.at[pl.ds(0, CH)], hist.at[pl.ds(r0 + 2 * CH, CH)])
    pltpu.sync_copy(obuf.at[pl.ds(0, CH)], hist.at[pl.ds(r0 + 3 * CH, CH)])
    pltpu.sync_copy(obuf.at[pl.ds(0, RPT - 4 * CH)],
                    hist.at[pl.ds(r0 + 4 * CH, RPT - 4 * CH)])

    one = jnp.ones((16,), jnp.float32)

    def fill(i, _):
        obuf[i, pl.ds(0, 16)] = one
        return 0

    lax.fori_loop(0, CH, fill, 0)
    plsc.subcore_barrier()

    def chunk(j, _):
        base = w * EPW + j * CH
        pltpu.sync_copy(dst_hbm.at[pl.ds(base, CH)], didx)
        pltpu.sync_copy(obuf, hist.at[didx], add=True)
        return 0

    lax.fori_loop(0, NCHUNK, chunk, 0)
    plsc.subcore_barrier()

    pltpu.sync_copy(hist.at[pl.ds(r0, RPT)], out_hbm.at[c, pl.ds(r0, RPT)])


@functools.partial(
    pl.kernel,
    out_type=jax.ShapeDtypeStruct((NC, ACC_ROWS, D), jnp.float32),
    mesh=_mesh,
    scratch_types=[
        pltpu.VMEM((CH,), jnp.int32),
        pltpu.VMEM((CH,), jnp.int32),
        pltpu.VMEM((CH,), jnp.int32),
        pltpu.VMEM((CH,), jnp.int32),
        pltpu.VMEM((CH, D), jnp.float32),
        pltpu.VMEM((CH, D), jnp.float32),
        pltpu.VMEM_SHARED((ACC_ROWS, D), jnp.float32),
        pltpu.SemaphoreType.DMA,
        pltpu.SemaphoreType.DMA,
        pltpu.SemaphoreType.DMA,
        pltpu.SemaphoreType.DMA,
        pltpu.SemaphoreType.DMA,
        pltpu.SemaphoreType.DMA,
    ],
)
def _sc_scatter(g_hbm, src_hbm, dst_hbm, out_hbm, sidx0, sidx1, didx0, didx1,
                rows0, rows1, acc, gsem0, gsem1, ssem0, ssem1, dsem0, dsem1):
    c = lax.axis_index("c")
    s = lax.axis_index("s")
    w = s * NC + c

    _zero_vmem_rows(rows0, CH, D)
    r0 = s * RPT
    pltpu.sync_copy(rows0.at[pl.ds(0, CH)], acc.at[pl.ds(r0, CH)])
    pltpu.sync_copy(rows0.at[pl.ds(0, CH)], acc.at[pl.ds(r0 + CH, CH)])
    pltpu.sync_copy(rows0.at[pl.ds(0, CH)], acc.at[pl.ds(r0 + 2 * CH, CH)])
    pltpu.sync_copy(rows0.at[pl.ds(0, CH)], acc.at[pl.ds(r0 + 3 * CH, CH)])
    pltpu.sync_copy(rows0.at[pl.ds(0, RPT - 4 * CH)],
                    acc.at[pl.ds(r0 + 4 * CH, RPT - 4 * CH)])
    plsc.subcore_barrier()

    base = w * EPW

    pltpu.sync_copy(src_hbm.at[pl.ds(base, CH)], sidx0)
    pltpu.sync_copy(dst_hbm.at[pl.ds(base, CH)], didx0)
    pltpu.sync_copy(src_hbm.at[pl.ds(base + CH, CH)], sidx1)
    pltpu.sync_copy(dst_hbm.at[pl.ds(base + CH, CH)], didx1)
    pltpu.async_copy(g_hbm.at[sidx0], rows0, gsem0)
    pltpu.async_copy(g_hbm.at[sidx1], rows1, gsem1)
    nhalf = NCHUNK // 2

    def body(i, _):
        for p, (sidx, didx, rows, gsem, ssem, dsem) in enumerate(
                ((sidx0, didx0, rows0, gsem0, ssem0, dsem0),
                 (sidx1, didx1, rows1, gsem1, ssem1, dsem1))):
            j = 2 * i + p
            pltpu.make_async_copy(g_hbm.at[sidx], rows, gsem).wait()

            pltpu.sync_copy(rows, acc.at[didx], add=True)

            @pl.when(i < nhalf - 1)
            def _():
                pltpu.sync_copy(src_hbm.at[pl.ds(base + (j + 2) * CH, CH)],
                                sidx)
                pltpu.sync_copy(dst_hbm.at[pl.ds(base + (j + 2) * CH, CH)],
                                didx)
                pltpu.async_copy(g_hbm.at[sidx], rows, gsem)

        return 0

    lax.fori_loop(0, nhalf, body, 0)
    plsc.subcore_barrier()

    pltpu.sync_copy(acc.at[pl.ds(r0, RPT)], out_hbm.at[c, pl.ds(r0, RPT)])


_RB = 1000
_GRID = N // _RB


def _tc_first_body(x_ref, w_ref, d0_ref, d1_ref, g_ref, dinv_ref):
    deg = 1.0 + d0_ref[:, 0] + d1_ref[:, 0]
    dinv = lax.rsqrt(deg)[:, None]
    g_ref[...] = jnp.dot(x_ref[...], w_ref[...],
                         preferred_element_type=jnp.float32) * dinv
    dinv_ref[...] = jnp.broadcast_to(dinv, dinv_ref.shape)


def _tc_first(x, w0, d0, d1):
    return pl.pallas_call(
        _tc_first_body,
        grid=(_GRID,),
        in_specs=[
            pl.BlockSpec((_RB, D), lambda i: (i, 0)),
            pl.BlockSpec((D, D), lambda i: (0, 0)),
            pl.BlockSpec((_RB, 16), lambda i: (i, 0)),
            pl.BlockSpec((_RB, 16), lambda i: (i, 0)),
        ],
        out_specs=[
            pl.BlockSpec((_RB, D), lambda i: (i, 0)),
            pl.BlockSpec((_RB, D), lambda i: (i, 0)),
        ],
        out_shape=[
            jax.ShapeDtypeStruct((N, D), jnp.float32),
            jax.ShapeDtypeStruct((N, D), jnp.float32),
        ],
    )(x, w0, d0, d1)


def _tc_mid_body(acc_ref, g_ref, dinv_ref, b_ref, w_ref, out_ref):
    a = acc_ref[0] + acc_ref[1] + g_ref[...]
    h = jnp.maximum(a * dinv_ref[...] + b_ref[...], 0.0)
    out_ref[...] = jnp.dot(h, w_ref[...],
                           preferred_element_type=jnp.float32) * dinv_ref[...]


def _tc_mid(accp, g, dinvb, b, w):
    return pl.pallas_call(
        _tc_mid_body,
        grid=(_GRID,),
        in_specs=[
            pl.BlockSpec((NC, _RB, D), lambda i: (0, i, 0)),
            pl.BlockSpec((_RB, D), lambda i: (i, 0)),
            pl.BlockSpec((_RB, D), lambda i: (i, 0)),
            pl.BlockSpec((1, D), lambda i: (0, 0)),
            pl.BlockSpec((D, D), lambda i: (0, 0)),
        ],
        out_specs=pl.BlockSpec((_RB, D), lambda i: (i, 0)),
        out_shape=jax.ShapeDtypeStruct((N, D), jnp.float32),
    )(accp, g, dinvb, b, w)


def _tc_final_body(acc_ref, g_ref, dinv_ref, b_ref, wfc_ref, bfc_ref, out_ref):
    a = acc_ref[0] + acc_ref[1] + g_ref[...]
    h = jnp.maximum(a * dinv_ref[...] + b_ref[...], 0.0)
    out_ref[...] = jnp.dot(h, wfc_ref[...],
                           preferred_element_type=jnp.float32) + bfc_ref[...]


def _tc_final(accp, g, dinvb, b, wfc, bfc):
    return pl.pallas_call(
        _tc_final_body,
        grid=(_GRID,),
        in_specs=[
            pl.BlockSpec((NC, _RB, D), lambda i: (0, i, 0)),
            pl.BlockSpec((_RB, D), lambda i: (i, 0)),
            pl.BlockSpec((_RB, D), lambda i: (i, 0)),
            pl.BlockSpec((1, D), lambda i: (0, 0)),
            pl.BlockSpec((D, NCLS), lambda i: (0, 0)),
            pl.BlockSpec((1, NCLS), lambda i: (0, 0)),
        ],
        out_specs=pl.BlockSpec((_RB, NCLS), lambda i: (i, 0)),
        out_shape=jax.ShapeDtypeStruct((N, NCLS), jnp.float32),
    )(accp, g, dinvb, b, wfc, bfc)


def kernel(x, edge_index, W0, b0, W1, b1, W2, b2, Wfc, bfc):
    src = edge_index[0].astype(jnp.int32)
    dst = edge_index[1].astype(jnp.int32)

    src_w = src.reshape(NW, E // NW)
    dst_w = dst.reshape(NW, E // NW)
    pad_s = jnp.zeros((NW, PAD), jnp.int32)
    pad_d = jnp.full((NW, PAD), TRASH, jnp.int32)
    src_p = jnp.concatenate([src_w, pad_s], axis=1).reshape(-1)
    dst_p = jnp.concatenate([dst_w, pad_d], axis=1).reshape(-1)

    degp = _sc_degree(dst_p)
    d0 = degp[0, :N, :]
    d1 = degp[1, :N, :]

    g0, dinvb = _tc_first(x, W0, d0, d1)

    acc1 = _sc_scatter(g0, src_p, dst_p)
    g1 = _tc_mid(acc1[:, :N, :], g0, dinvb, b0.reshape(1, D), W1)

    acc2 = _sc_scatter(g1, src_p, dst_p)
    g2 = _tc_mid(acc2[:, :N, :], g1, dinvb, b1.reshape(1, D), W2)

    acc3 = _sc_scatter(g2, src_p, dst_p)
    out = _tc_final(acc3[:, :N, :], g2, dinvb, b2.reshape(1, D), Wfc,
                    bfc.reshape(1, NCLS))
    return out
